# Initial kernel scaffold; baseline (speedup 1.0000x reference)
import jax
import jax.numpy as jnp
from jax.experimental import pallas as pl


def kernel(x, edge_index, W1, b1, W2, b2):
    raise NotImplementedError("write your pallas kernel here")


# speedup vs baseline: 8.1728x; 8.1728x over previous
import functools

import jax
import jax.numpy as jnp
from jax import lax
from jax.experimental import pallas as pl
from jax.experimental.pallas import tpu as pltpu
from jax.experimental.pallas import tpu_sc as plsc

N = 10000
E = 320000
D_IN = 128
D_HID = 256
D_OUT = 128

NC = 2
NS = 16
B = 128

R_DEG = 80
R_TOT = NC * NS * R_DEG
E_PAD = R_TOT * B
R_SCAT = R_TOT // NS
N_TAB = N + 16

CH = 80
NCH = N // CH
CPT = (NCH + NS - 1) // NS

CHS = 16
NCHS = N // CHS
CPTS = (NCHS + NS - 1) // NS
IGRP = 8
NGRP = R_SCAT // IGRP
R_SC2 = R_TOT // (NC * NS)
NGRP2 = R_SC2 // IGRP


def _mesh():
    return plsc.VectorSubcoreMesh(
        core_axis_name="c", subcore_axis_name="s", num_cores=NC, num_subcores=NS
    )


def _chunked(s, copy_fn, n_chunks, max_per_tile, rows):
    @pl.loop(0, max_per_tile)
    def _(k):
        cid = s + k * NS

        @pl.when(cid < n_chunks)
        def _():
            copy_fn(pl.multiple_of(cid * rows, 8))


def _deg_body(dst_hbm, zeros_hbm, ones_hbm, out0_hbm, out1_hbm,
              deg_acc, didx, ones_v, stage):
    c = lax.axis_index("c")
    s = lax.axis_index("s")
    w = c * NS + s
    pltpu.sync_copy(dst_hbm.at[pl.ds(pl.multiple_of(w * R_DEG, 8), R_DEG)], didx)
    pltpu.sync_copy(ones_hbm, ones_v)

    pltpu.sync_copy(zeros_hbm, stage)

    def zinit(off):
        pltpu.sync_copy(stage, deg_acc.at[pl.ds(off, CH)])
    _chunked(s, zinit, NCH, CPT, CH)

    plsc.subcore_barrier()

    @pl.loop(0, R_DEG)
    def _(j):
        pltpu.sync_copy(ones_v, deg_acc.at[didx.at[j]], add=True)

    plsc.subcore_barrier()

    def wb(out_hbm):
        def cp(off):
            pltpu.sync_copy(deg_acc.at[pl.ds(off, CH)], stage)
            pltpu.sync_copy(stage, out_hbm.at[pl.ds(off, CH)])
        _chunked(s, cp, NCH, CPT, CH)

    @pl.when(c == 0)
    def _():
        wb(out0_hbm)

    @pl.when(c == 1)
    def _():
        wb(out1_hbm)


def _deg_kernel(dstp, zeros, ones):
    kfn = pl.kernel(
        _deg_body,
        out_type=[
            jax.ShapeDtypeStruct((N,), jnp.float32),
            jax.ShapeDtypeStruct((N,), jnp.float32),
        ],
        mesh=_mesh(),
        scratch_types=[
            pltpu.VMEM_SHARED((N_TAB,), jnp.float32),
            pltpu.VMEM((R_DEG, B), jnp.int32),
            pltpu.VMEM((B,), jnp.float32),
            pltpu.VMEM((CH,), jnp.float32),
        ],
    )
    return kfn(dstp, zeros, ones)


def _scatter_body(dh, ylo, yhi, src_hbm, dst_hbm, out_hbm, acc, sidx, didx,
                  rowbuf, stage, sem):
    c = lax.axis_index("c")
    s = lax.axis_index("s")

    def init_and_edges(table):
        def cp(off):
            pltpu.sync_copy(table.at[pl.ds(off, CHS)], stage)
            pltpu.sync_copy(stage, acc.at[pl.ds(off, CHS)])
        _chunked(s, cp, NCHS, CPTS, CHS)
        plsc.subcore_barrier()

        @pl.loop(0, NGRP)
        def _(g):
            off_i = pl.multiple_of(s * R_SCAT + g * IGRP, 8)
            pltpu.sync_copy(src_hbm.at[pl.ds(off_i, IGRP)], sidx)
            pltpu.sync_copy(dst_hbm.at[pl.ds(off_i, IGRP)], didx)

            @pl.loop(0, IGRP)
            def _(j):
                pltpu.async_copy(table.at[sidx.at[j]], rowbuf, sem).wait()
                pltpu.sync_copy(rowbuf, acc.at[didx.at[j]], add=True)

    @pl.when(c == 0)
    def _():
        init_and_edges(ylo)

    @pl.when(c == 1)
    def _():
        init_and_edges(yhi)

    plsc.subcore_barrier()

    def wb(off):
        pltpu.sync_copy(acc.at[pl.ds(off, CHS)], stage)
        pltpu.sync_copy(stage, out_hbm.at[c, pl.ds(off, CHS)])
    _chunked(s, wb, NCHS, CPTS, CHS)


def _scatter_kernel(dh, ylo, yhi, srcp, dstp):
    kfn = pl.kernel(
        functools.partial(_scatter_body, dh),
        out_type=jax.ShapeDtypeStruct((NC, N, dh), jnp.float32),
        mesh=_mesh(),
        scratch_types=[
            pltpu.VMEM_SHARED((N_TAB, dh), jnp.float32),
            pltpu.VMEM((IGRP, B), jnp.int32),
            pltpu.VMEM((IGRP, B), jnp.int32),
            pltpu.VMEM((B, dh), jnp.float32),
            pltpu.VMEM((CHS, dh), jnp.float32),
            pltpu.SemaphoreType.DMA,
        ],
    )
    return kfn(ylo, yhi, srcp, dstp)


def _scatter2_body(y, zeros2_hbm, src_hbm, dst_hbm, out_hbm, acc, sidx, didx,
                   rowbuf, stage, sem):
    c = lax.axis_index("c")
    s = lax.axis_index("s")

    @pl.when(c == 0)
    def _():
        def cp(off):
            pltpu.sync_copy(y.at[pl.ds(off, CHS)], stage)
            pltpu.sync_copy(stage, acc.at[pl.ds(off, CHS)])
        _chunked(s, cp, NCHS, CPTS, CHS)

    @pl.when(c == 1)
    def _():
        pltpu.sync_copy(zeros2_hbm, stage)

        def cp(off):
            pltpu.sync_copy(stage, acc.at[pl.ds(off, CHS)])
        _chunked(s, cp, NCHS, CPTS, CHS)

    plsc.subcore_barrier()

    @pl.loop(0, NGRP2)
    def _(g):
        off_i = pl.multiple_of((c * NS + s) * R_SC2 + g * IGRP, 8)
        pltpu.sync_copy(src_hbm.at[pl.ds(off_i, IGRP)], sidx)
        pltpu.sync_copy(dst_hbm.at[pl.ds(off_i, IGRP)], didx)

        @pl.loop(0, IGRP)
        def _(j):
            pltpu.async_copy(y.at[sidx.at[j]], rowbuf, sem).wait()
            pltpu.sync_copy(rowbuf, acc.at[didx.at[j]], add=True)

    plsc.subcore_barrier()

    def wb(off):
        pltpu.sync_copy(acc.at[pl.ds(off, CHS)], stage)
        pltpu.sync_copy(stage, out_hbm.at[c, pl.ds(off, CHS)])
    _chunked(s, wb, NCHS, CPTS, CHS)


def _scatter2_kernel(y, zeros2, srcp, dstp):
    kfn = pl.kernel(
        _scatter2_body,
        out_type=jax.ShapeDtypeStruct((NC, N, D_OUT), jnp.float32),
        mesh=_mesh(),
        scratch_types=[
            pltpu.VMEM_SHARED((N_TAB, D_OUT), jnp.float32),
            pltpu.VMEM((IGRP, B), jnp.int32),
            pltpu.VMEM((IGRP, B), jnp.int32),
            pltpu.VMEM((B, D_OUT), jnp.float32),
            pltpu.VMEM((CHS, D_OUT), jnp.float32),
            pltpu.SemaphoreType.DMA,
        ],
    )
    return kfn(y, zeros2, srcp, dstp)


def _dis(deg0_ref, deg1_ref):
    deg = deg0_ref[...] + deg1_ref[...] + 1.0
    return lax.rsqrt(deg)[:, None]


def _tc1_body(x_ref, w_ref, deg0_ref, deg1_ref, ylo_ref, yhi_ref):
    y = jnp.dot(x_ref[...], w_ref[...], preferred_element_type=jnp.float32)
    y = y * _dis(deg0_ref, deg1_ref)
    ylo_ref[...] = y[:, : D_HID // 2]
    yhi_ref[...] = y[:, D_HID // 2:]


def _tc1(x, W1, deg0, deg1):
    return pl.pallas_call(
        _tc1_body,
        out_shape=[
            jax.ShapeDtypeStruct((N, D_HID // 2), jnp.float32),
            jax.ShapeDtypeStruct((N, D_HID // 2), jnp.float32),
        ],
    )(x, W1, deg0, deg1)


def _tc2_body(acc_ref, deg0_ref, deg1_ref, b1_ref, w2_ref, y_ref):
    dis = _dis(deg0_ref, deg1_ref)
    acc = jnp.concatenate([acc_ref[0], acc_ref[1]], axis=1)
    h = jnp.maximum(acc * dis + b1_ref[...][None, :], 0.0)
    y_ref[...] = jnp.dot(h, w2_ref[...],
                         preferred_element_type=jnp.float32) * dis


def _tc2(acc1, deg0, deg1, b1, W2):
    return pl.pallas_call(
        _tc2_body,
        out_shape=jax.ShapeDtypeStruct((N, D_OUT), jnp.float32),
    )(acc1, deg0, deg1, b1, W2)


def _tc3_body(acc_ref, deg0_ref, deg1_ref, b2_ref, out_ref):
    acc = acc_ref[0] + acc_ref[1]
    out_ref[...] = jnp.maximum(
        acc * _dis(deg0_ref, deg1_ref) + b2_ref[...][None, :], 0.0)


def _tc3(acc2, deg0, deg1, b2):
    return pl.pallas_call(
        _tc3_body,
        out_shape=jax.ShapeDtypeStruct((N, D_OUT), jnp.float32),
    )(acc2, deg0, deg1, b2)


def kernel(x, edge_index, W1, b1, W2, b2):
    ei = edge_index.astype(jnp.int32)
    pad = E_PAD - E
    srcp = jnp.concatenate(
        [ei[0], jnp.zeros((pad,), jnp.int32)]).reshape(R_TOT, B)
    dstp = jnp.concatenate(
        [ei[1], jnp.full((pad,), N, jnp.int32)]).reshape(R_TOT, B)
    zeros = jnp.zeros((CH,), jnp.float32)
    zeros2 = jnp.zeros((CHS, D_OUT), jnp.float32)
    ones = jnp.ones((B,), jnp.float32)

    deg0, deg1 = _deg_kernel(dstp, zeros, ones)
    y1lo, y1hi = _tc1(x, W1, deg0, deg1)
    acc1 = _scatter_kernel(D_HID // 2, y1lo, y1hi, srcp, dstp)
    y2 = _tc2(acc1, deg0, deg1, b1, W2)
    acc2 = _scatter2_kernel(y2, zeros2, srcp, dstp)
    return _tc3(acc2, deg0, deg1, b2)


# speedup vs baseline: 9.1853x; 1.1239x over previous
import functools

import jax
import jax.numpy as jnp
from jax import lax
from jax.experimental import pallas as pl
from jax.experimental.pallas import tpu as pltpu
from jax.experimental.pallas import tpu_sc as plsc

N = 10000
E = 320000
D_IN = 128
D_HID = 256
D_OUT = 128

NC = 2
NS = 16
B = 128

R_DEG = 80
R_TOT = NC * NS * R_DEG
E_PAD = R_TOT * B
R_SCAT = R_TOT // NS
N_TAB = N + 16

CH = 80
NCH = N // CH
CPT = (NCH + NS - 1) // NS

CHS = 16
NCHS = N // CHS
CPTS = (NCHS + NS - 1) // NS
IGRP = 8
NGRP = R_SCAT // IGRP
R_SC2 = R_TOT // (NC * NS)
NGRP2 = R_SC2 // IGRP


def _mesh():
    return plsc.VectorSubcoreMesh(
        core_axis_name="c", subcore_axis_name="s", num_cores=NC, num_subcores=NS
    )


def _edge_loop(table, src_hbm, dst_hbm, acc, sidx, didx, buf0, buf1,
               sem0, sem1, base_rows, ngrp):
    bufs = (buf0, buf1)
    sems = (sem0, sem1)

    @pl.loop(0, ngrp)
    def _(g):
        off_i = pl.multiple_of(base_rows + g * IGRP, 8)
        pltpu.sync_copy(src_hbm.at[pl.ds(off_i, IGRP)], sidx)
        pltpu.sync_copy(dst_hbm.at[pl.ds(off_i, IGRP)], didx)
        descs = [None] * IGRP
        descs[0] = pltpu.async_copy(table.at[sidx.at[0]], bufs[0], sems[0])
        for j in range(IGRP):
            if j + 1 < IGRP:
                descs[j + 1] = pltpu.async_copy(
                    table.at[sidx.at[j + 1]], bufs[(j + 1) % 2],
                    sems[(j + 1) % 2])
            descs[j].wait()
            pltpu.sync_copy(bufs[j % 2], acc.at[didx.at[j]], add=True)


def _chunked(s, copy_fn, n_chunks, max_per_tile, rows):
    @pl.loop(0, max_per_tile)
    def _(k):
        cid = s + k * NS

        @pl.when(cid < n_chunks)
        def _():
            copy_fn(pl.multiple_of(cid * rows, 8))


def _deg_body(dst_hbm, zeros_hbm, ones_hbm, out0_hbm, out1_hbm,
              deg_acc, didx, ones_v, stage):
    c = lax.axis_index("c")
    s = lax.axis_index("s")
    w = c * NS + s
    pltpu.sync_copy(dst_hbm.at[pl.ds(pl.multiple_of(w * R_DEG, 8), R_DEG)], didx)
    pltpu.sync_copy(ones_hbm, ones_v)

    pltpu.sync_copy(zeros_hbm, stage)

    def zinit(off):
        pltpu.sync_copy(stage, deg_acc.at[pl.ds(off, CH)])
    _chunked(s, zinit, NCH, CPT, CH)

    plsc.subcore_barrier()

    @pl.loop(0, R_DEG)
    def _(j):
        pltpu.sync_copy(ones_v, deg_acc.at[didx.at[j]], add=True)

    plsc.subcore_barrier()

    def wb(out_hbm):
        def cp(off):
            pltpu.sync_copy(deg_acc.at[pl.ds(off, CH)], stage)
            pltpu.sync_copy(stage, out_hbm.at[pl.ds(off, CH)])
        _chunked(s, cp, NCH, CPT, CH)

    @pl.when(c == 0)
    def _():
        wb(out0_hbm)

    @pl.when(c == 1)
    def _():
        wb(out1_hbm)


def _deg_kernel(dstp, zeros, ones):
    kfn = pl.kernel(
        _deg_body,
        out_type=[
            jax.ShapeDtypeStruct((N,), jnp.float32),
            jax.ShapeDtypeStruct((N,), jnp.float32),
        ],
        mesh=_mesh(),
        scratch_types=[
            pltpu.VMEM_SHARED((N_TAB,), jnp.float32),
            pltpu.VMEM((R_DEG, B), jnp.int32),
            pltpu.VMEM((B,), jnp.float32),
            pltpu.VMEM((CH,), jnp.float32),
        ],
    )
    return kfn(dstp, zeros, ones)


def _scatter_body(dh, ylo, yhi, src_hbm, dst_hbm, out_hbm, acc, sidx, didx,
                  buf0, buf1, stage, sem0, sem1):
    c = lax.axis_index("c")
    s = lax.axis_index("s")

    def init_and_edges(table):
        def cp(off):
            pltpu.sync_copy(table.at[pl.ds(off, CHS)], stage)
            pltpu.sync_copy(stage, acc.at[pl.ds(off, CHS)])
        _chunked(s, cp, NCHS, CPTS, CHS)
        plsc.subcore_barrier()

        _edge_loop(table, src_hbm, dst_hbm, acc, sidx, didx, buf0, buf1,
                   sem0, sem1, s * R_SCAT, NGRP)

    @pl.when(c == 0)
    def _():
        init_and_edges(ylo)

    @pl.when(c == 1)
    def _():
        init_and_edges(yhi)

    plsc.subcore_barrier()

    def wb(off):
        pltpu.sync_copy(acc.at[pl.ds(off, CHS)], stage)
        pltpu.sync_copy(stage, out_hbm.at[c, pl.ds(off, CHS)])
    _chunked(s, wb, NCHS, CPTS, CHS)


def _scatter_kernel(dh, ylo, yhi, srcp, dstp):
    kfn = pl.kernel(
        functools.partial(_scatter_body, dh),
        out_type=jax.ShapeDtypeStruct((NC, N, dh), jnp.float32),
        mesh=_mesh(),
        scratch_types=[
            pltpu.VMEM_SHARED((N_TAB, dh), jnp.float32),
            pltpu.VMEM((IGRP, B), jnp.int32),
            pltpu.VMEM((IGRP, B), jnp.int32),
            pltpu.VMEM((B, dh), jnp.float32),
            pltpu.VMEM((B, dh), jnp.float32),
            pltpu.VMEM((CHS, dh), jnp.float32),
            pltpu.SemaphoreType.DMA,
            pltpu.SemaphoreType.DMA,
        ],
    )
    return kfn(ylo, yhi, srcp, dstp)


def _scatter2_body(y, zeros2_hbm, src_hbm, dst_hbm, out_hbm, acc, sidx, didx,
                   buf0, buf1, stage, sem0, sem1):
    c = lax.axis_index("c")
    s = lax.axis_index("s")

    @pl.when(c == 0)
    def _():
        def cp(off):
            pltpu.sync_copy(y.at[pl.ds(off, CHS)], stage)
            pltpu.sync_copy(stage, acc.at[pl.ds(off, CHS)])
        _chunked(s, cp, NCHS, CPTS, CHS)

    @pl.when(c == 1)
    def _():
        pltpu.sync_copy(zeros2_hbm, stage)

        def cp(off):
            pltpu.sync_copy(stage, acc.at[pl.ds(off, CHS)])
        _chunked(s, cp, NCHS, CPTS, CHS)

    plsc.subcore_barrier()

    _edge_loop(y, src_hbm, dst_hbm, acc, sidx, didx, buf0, buf1,
               sem0, sem1, (c * NS + s) * R_SC2, NGRP2)

    plsc.subcore_barrier()

    def wb(off):
        pltpu.sync_copy(acc.at[pl.ds(off, CHS)], stage)
        pltpu.sync_copy(stage, out_hbm.at[c, pl.ds(off, CHS)])
    _chunked(s, wb, NCHS, CPTS, CHS)


def _scatter2_kernel(y, zeros2, srcp, dstp):
    kfn = pl.kernel(
        _scatter2_body,
        out_type=jax.ShapeDtypeStruct((NC, N, D_OUT), jnp.float32),
        mesh=_mesh(),
        scratch_types=[
            pltpu.VMEM_SHARED((N_TAB, D_OUT), jnp.float32),
            pltpu.VMEM((IGRP, B), jnp.int32),
            pltpu.VMEM((IGRP, B), jnp.int32),
            pltpu.VMEM((B, D_OUT), jnp.float32),
            pltpu.VMEM((B, D_OUT), jnp.float32),
            pltpu.VMEM((CHS, D_OUT), jnp.float32),
            pltpu.SemaphoreType.DMA,
            pltpu.SemaphoreType.DMA,
        ],
    )
    return kfn(y, zeros2, srcp, dstp)


def _dis(deg0_ref, deg1_ref):
    deg = deg0_ref[...] + deg1_ref[...] + 1.0
    return lax.rsqrt(deg)[:, None]


def _tc1_body(x_ref, w_ref, deg0_ref, deg1_ref, ylo_ref, yhi_ref):
    y = jnp.dot(x_ref[...], w_ref[...], preferred_element_type=jnp.float32)
    y = y * _dis(deg0_ref, deg1_ref)
    ylo_ref[...] = y[:, : D_HID // 2]
    yhi_ref[...] = y[:, D_HID // 2:]


def _tc1(x, W1, deg0, deg1):
    return pl.pallas_call(
        _tc1_body,
        out_shape=[
            jax.ShapeDtypeStruct((N, D_HID // 2), jnp.float32),
            jax.ShapeDtypeStruct((N, D_HID // 2), jnp.float32),
        ],
    )(x, W1, deg0, deg1)


def _tc2_body(acc_ref, deg0_ref, deg1_ref, b1_ref, w2_ref, y_ref):
    dis = _dis(deg0_ref, deg1_ref)
    acc = jnp.concatenate([acc_ref[0], acc_ref[1]], axis=1)
    h = jnp.maximum(acc * dis + b1_ref[...][None, :], 0.0)
    y_ref[...] = jnp.dot(h, w2_ref[...],
                         preferred_element_type=jnp.float32) * dis


def _tc2(acc1, deg0, deg1, b1, W2):
    return pl.pallas_call(
        _tc2_body,
        out_shape=jax.ShapeDtypeStruct((N, D_OUT), jnp.float32),
    )(acc1, deg0, deg1, b1, W2)


def _tc3_body(acc_ref, deg0_ref, deg1_ref, b2_ref, out_ref):
    acc = acc_ref[0] + acc_ref[1]
    out_ref[...] = jnp.maximum(
        acc * _dis(deg0_ref, deg1_ref) + b2_ref[...][None, :], 0.0)


def _tc3(acc2, deg0, deg1, b2):
    return pl.pallas_call(
        _tc3_body,
        out_shape=jax.ShapeDtypeStruct((N, D_OUT), jnp.float32),
    )(acc2, deg0, deg1, b2)


def kernel(x, edge_index, W1, b1, W2, b2):
    ei = edge_index.astype(jnp.int32)
    pad = E_PAD - E
    srcp = jnp.concatenate(
        [ei[0], jnp.zeros((pad,), jnp.int32)]).reshape(R_TOT, B)
    dstp = jnp.concatenate(
        [ei[1], jnp.full((pad,), N, jnp.int32)]).reshape(R_TOT, B)
    zeros = jnp.zeros((CH,), jnp.float32)
    zeros2 = jnp.zeros((CHS, D_OUT), jnp.float32)
    ones = jnp.ones((B,), jnp.float32)

    deg0, deg1 = _deg_kernel(dstp, zeros, ones)
    y1lo, y1hi = _tc1(x, W1, deg0, deg1)
    acc1 = _scatter_kernel(D_HID // 2, y1lo, y1hi, srcp, dstp)
    y2 = _tc2(acc1, deg0, deg1, b1, W2)
    acc2 = _scatter2_kernel(y2, zeros2, srcp, dstp)
    return _tc3(acc2, deg0, deg1, b2)


# speedup vs baseline: 10.2941x; 1.1207x over previous
import jax
import jax.numpy as jnp
from jax import lax
from jax.experimental import pallas as pl
from jax.experimental.pallas import tpu as pltpu
from jax.experimental.pallas import tpu_sc as plsc

N = 10000
E = 320000
D_IN = 128
D_HID = 256
D_OUT = 128
D = 128

NC = 2
NS = 16
B = 128

R_DEG = 80
R_TOT = NC * NS * R_DEG
E_PAD = R_TOT * B
N_TAB = N + 16

CH = 80
NCH = N // CH
CPT = (NCH + NS - 1) // NS

CHS = 16
NCHS = N // CHS
CPTS = (NCHS + NS - 1) // NS
IGRP = 8
R_SC = R_TOT // (NC * NS)
NGRP = R_SC // IGRP


def _mesh():
    return plsc.VectorSubcoreMesh(
        core_axis_name="c", subcore_axis_name="s", num_cores=NC, num_subcores=NS
    )


def _edge_loop(table, src_hbm, dst_hbm, acc, sidx, didx, buf0, buf1,
               sem0, sem1, base_rows, ngrp):
    bufs = (buf0, buf1)
    sems = (sem0, sem1)

    @pl.loop(0, ngrp)
    def _(g):
        off_i = pl.multiple_of(base_rows + g * IGRP, 8)
        pltpu.sync_copy(src_hbm.at[pl.ds(off_i, IGRP)], sidx)
        pltpu.sync_copy(dst_hbm.at[pl.ds(off_i, IGRP)], didx)
        descs = [None] * IGRP
        descs[0] = pltpu.async_copy(table.at[sidx.at[0]], bufs[0], sems[0])
        for j in range(IGRP):
            if j + 1 < IGRP:
                descs[j + 1] = pltpu.async_copy(
                    table.at[sidx.at[j + 1]], bufs[(j + 1) % 2],
                    sems[(j + 1) % 2])
            descs[j].wait()
            pltpu.sync_copy(bufs[j % 2], acc.at[didx.at[j]], add=True)


def _chunked(s, copy_fn, n_chunks, max_per_tile, rows):
    @pl.loop(0, max_per_tile)
    def _(k):
        cid = s + k * NS

        @pl.when(cid < n_chunks)
        def _():
            copy_fn(pl.multiple_of(cid * rows, 8))


def _deg_body(dst_hbm, zeros_hbm, ones_hbm, out0_hbm, out1_hbm,
              deg_acc, didx, ones_v, stage):
    c = lax.axis_index("c")
    s = lax.axis_index("s")
    w = c * NS + s
    pltpu.sync_copy(dst_hbm.at[pl.ds(pl.multiple_of(w * R_DEG, 8), R_DEG)], didx)
    pltpu.sync_copy(ones_hbm, ones_v)

    pltpu.sync_copy(zeros_hbm, stage)

    def zinit(off):
        pltpu.sync_copy(stage, deg_acc.at[pl.ds(off, CH)])
    _chunked(s, zinit, NCH, CPT, CH)

    plsc.subcore_barrier()

    @pl.loop(0, R_DEG)
    def _(j):
        pltpu.sync_copy(ones_v, deg_acc.at[didx.at[j]], add=True)

    plsc.subcore_barrier()

    def wb(out_hbm):
        def cp(off):
            pltpu.sync_copy(deg_acc.at[pl.ds(off, CH)], stage)
            pltpu.sync_copy(stage, out_hbm.at[pl.ds(off, CH)])
        _chunked(s, cp, NCH, CPT, CH)

    @pl.when(c == 0)
    def _():
        wb(out0_hbm)

    @pl.when(c == 1)
    def _():
        wb(out1_hbm)


def _deg_kernel(dstp, zeros, ones):
    kfn = pl.kernel(
        _deg_body,
        out_type=[
            jax.ShapeDtypeStruct((N,), jnp.float32),
            jax.ShapeDtypeStruct((N,), jnp.float32),
        ],
        mesh=_mesh(),
        scratch_types=[
            pltpu.VMEM_SHARED((N_TAB,), jnp.float32),
            pltpu.VMEM((R_DEG, B), jnp.int32),
            pltpu.VMEM((B,), jnp.float32),
            pltpu.VMEM((CH,), jnp.float32),
        ],
    )
    return kfn(dstp, zeros, ones)


def _scatter_body(y, zeros2_hbm, src_hbm, dst_hbm, out_hbm, acc, sidx, didx,
                  buf0, buf1, stage, sem0, sem1):
    c = lax.axis_index("c")
    s = lax.axis_index("s")

    @pl.when(c == 0)
    def _():
        def cp(off):
            pltpu.sync_copy(y.at[pl.ds(off, CHS)], stage)
            pltpu.sync_copy(stage, acc.at[pl.ds(off, CHS)])
        _chunked(s, cp, NCHS, CPTS, CHS)

    @pl.when(c == 1)
    def _():
        pltpu.sync_copy(zeros2_hbm, stage)

        def cp(off):
            pltpu.sync_copy(stage, acc.at[pl.ds(off, CHS)])
        _chunked(s, cp, NCHS, CPTS, CHS)

    plsc.subcore_barrier()

    _edge_loop(y, src_hbm, dst_hbm, acc, sidx, didx, buf0, buf1,
               sem0, sem1, (c * NS + s) * R_SC, NGRP)

    plsc.subcore_barrier()

    def wb(off):
        pltpu.sync_copy(acc.at[pl.ds(off, CHS)], stage)
        pltpu.sync_copy(stage, out_hbm.at[c, pl.ds(off, CHS)])
    _chunked(s, wb, NCHS, CPTS, CHS)


def _scatter_kernel(y, zeros2, srcp, dstp):
    kfn = pl.kernel(
        _scatter_body,
        out_type=jax.ShapeDtypeStruct((NC, N, D), jnp.float32),
        mesh=_mesh(),
        scratch_types=[
            pltpu.VMEM_SHARED((N_TAB, D), jnp.float32),
            pltpu.VMEM((IGRP, B), jnp.int32),
            pltpu.VMEM((IGRP, B), jnp.int32),
            pltpu.VMEM((B, D), jnp.float32),
            pltpu.VMEM((B, D), jnp.float32),
            pltpu.VMEM((CHS, D), jnp.float32),
            pltpu.SemaphoreType.DMA,
            pltpu.SemaphoreType.DMA,
        ],
    )
    return kfn(y, zeros2, srcp, dstp)


def _dis(deg0_ref, deg1_ref):
    deg = deg0_ref[...] + deg1_ref[...] + 1.0
    return lax.rsqrt(deg)[:, None]


def _tca_body(x_ref, deg0_ref, deg1_ref, xt_ref):
    xt_ref[...] = x_ref[...] * _dis(deg0_ref, deg1_ref)


def _tca(x, deg0, deg1):
    return pl.pallas_call(
        _tca_body,
        out_shape=jax.ShapeDtypeStruct((N, D_IN), jnp.float32),
    )(x, deg0, deg1)


def _tcb_body(agg_ref, deg0_ref, deg1_ref, w1_ref, b1_ref, w2_ref, y_ref):
    dis = _dis(deg0_ref, deg1_ref)
    agg = (agg_ref[0] + agg_ref[1]) * dis
    h = jnp.maximum(
        jnp.dot(agg, w1_ref[...], preferred_element_type=jnp.float32)
        + b1_ref[...][None, :], 0.0)
    y_ref[...] = jnp.dot(h, w2_ref[...],
                         preferred_element_type=jnp.float32) * dis


def _tcb(agg1, deg0, deg1, W1, b1, W2):
    return pl.pallas_call(
        _tcb_body,
        out_shape=jax.ShapeDtypeStruct((N, D_OUT), jnp.float32),
    )(agg1, deg0, deg1, W1, b1, W2)


def _tcc_body(agg_ref, deg0_ref, deg1_ref, b2_ref, out_ref):
    agg = agg_ref[0] + agg_ref[1]
    out_ref[...] = jnp.maximum(
        agg * _dis(deg0_ref, deg1_ref) + b2_ref[...][None, :], 0.0)


def _tcc(agg2, deg0, deg1, b2):
    return pl.pallas_call(
        _tcc_body,
        out_shape=jax.ShapeDtypeStruct((N, D_OUT), jnp.float32),
    )(agg2, deg0, deg1, b2)


def kernel(x, edge_index, W1, b1, W2, b2):
    ei = edge_index.astype(jnp.int32)
    pad = E_PAD - E
    srcp = jnp.concatenate(
        [ei[0], jnp.zeros((pad,), jnp.int32)]).reshape(R_TOT, B)
    dstp = jnp.concatenate(
        [ei[1], jnp.full((pad,), N, jnp.int32)]).reshape(R_TOT, B)
    zeros = jnp.zeros((CH,), jnp.float32)
    zeros2 = jnp.zeros((CHS, D), jnp.float32)
    ones = jnp.ones((B,), jnp.float32)

    deg0, deg1 = _deg_kernel(dstp, zeros, ones)
    xt = _tca(x, deg0, deg1)
    agg1 = _scatter_kernel(xt, zeros2, srcp, dstp)
    y2 = _tcb(agg1, deg0, deg1, W1, b1, W2)
    agg2 = _scatter_kernel(y2, zeros2, srcp, dstp)
    return _tcc(agg2, deg0, deg1, b2)
